# Initial kernel scaffold; baseline (speedup 1.0000x reference)
import jax
import jax.numpy as jnp
from jax.experimental import pallas as pl


def kernel(x, table):
    raise NotImplementedError("write your pallas kernel here")


# speedup vs baseline: 2.5967x; 2.5967x over previous
import functools

import jax
import jax.numpy as jnp
from jax import lax
from jax.experimental import pallas as pl
from jax.experimental.pallas import tpu as pltpu
from jax.experimental.pallas import tpu_sc as plsc

_NUM_CORES = 2
_NUM_SUBCORES = 16
_NUM_WORKERS = _NUM_CORES * _NUM_SUBCORES
_CHUNK_ROWS = 64
_NBUF = 2


def _copy_body(table_hbm, out_hbm, bufs, sems, *, rows_per_w, d):
    wid = lax.axis_index("s") * _NUM_CORES + lax.axis_index("c")
    base = wid * rows_per_w
    n_chunks = rows_per_w // _CHUNK_ROWS

    for b in range(min(_NBUF, n_chunks)):
        pltpu.make_async_copy(
            table_hbm.at[pl.ds(base + b * _CHUNK_ROWS, _CHUNK_ROWS), :],
            bufs[b],
            sems[b],
        ).start()

    def step(i, carry):
        for b in range(_NBUF):
            @pl.when(lax.rem(i, _NBUF) == b)
            def _():
                row = base + i * _CHUNK_ROWS
                pltpu.make_async_copy(
                    table_hbm.at[pl.ds(row, _CHUNK_ROWS), :],
                    bufs[b],
                    sems[b],
                ).wait()
                pltpu.sync_copy(bufs[b], out_hbm.at[pl.ds(row, _CHUNK_ROWS), :])
                nxt = i + _NBUF
                @pl.when(nxt < n_chunks)
                def _():
                    pltpu.make_async_copy(
                        table_hbm.at[pl.ds(base + nxt * _CHUNK_ROWS, _CHUNK_ROWS), :],
                        bufs[b],
                        sems[b],
                    ).start()
        return carry

    lax.fori_loop(0, n_chunks, step, 0)


def kernel(x, table):
    seq_len = x.shape[1]
    d = table.shape[1]
    rows_per_w = seq_len // _NUM_WORKERS
    mesh = plsc.VectorSubcoreMesh(core_axis_name="c", subcore_axis_name="s")

    k = pl.kernel(
        functools.partial(_copy_body, rows_per_w=rows_per_w, d=d),
        out_type=jax.ShapeDtypeStruct((seq_len, d), jnp.float32),
        mesh=mesh,
        scratch_types=[
            [pltpu.VMEM((_CHUNK_ROWS, d), jnp.float32) for _ in range(_NBUF)],
            [pltpu.SemaphoreType.DMA for _ in range(_NBUF)],
        ],
    )
    return k(table)
